# Initial kernel scaffold; baseline (speedup 1.0000x reference)
import functools

import jax
import jax.numpy as jnp
from jax import lax
from jax.experimental import pallas as pl
from jax.experimental.pallas import tpu as pltpu
from jax.experimental.pallas import tpu_sc as plsc

_N = 10000
_E = 320000
_F = 128
_H = 256
_C = 40

_NC = 2
_NS = 16
_CHUNK = 128
_EPAD = 323584
_ECHUNKS_AGG = _EPAD // (_NS * _CHUNK)
_EPW_AGG = _EPAD // _NS
_ECHUNKS_DEG = _EPAD // (_NC * _NS * _CHUNK)
_EPW_DEG = _EPAD // (_NC * _NS)
_NR = _N + 16
_NB_DEG = 10240
_ROWS_PER_SUB = _N // _NS
_INIT_CHUNK = 125

_BN = 1000
_NBLK = _N // _BN


_sc_mesh = plsc.VectorSubcoreMesh(core_axis_name="c", subcore_axis_name="s")


@functools.partial(
    pl.kernel,
    out_type=jax.ShapeDtypeStruct((_NC * _NB_DEG,), jnp.float32),
    mesh=_sc_mesh,
    scratch_types=[
        pltpu.VMEM((_CHUNK,), jnp.int32),
        pltpu.VMEM((_CHUNK,), jnp.float32),
        pltpu.VMEM((640,), jnp.float32),
        pltpu.VMEM_SHARED((_NB_DEG,), jnp.float32),
    ],
)
def _deg(dst_hbm, out_hbm, dstb, onesb, zb, acc):
    c = lax.axis_index("c")
    s = lax.axis_index("s")
    w = c * _NS + s
    for i in range(640 // 16):
        zb[pl.ds(i * 16, 16)] = jnp.zeros((16,), jnp.float32)
    for i in range(_CHUNK // 16):
        onesb[pl.ds(i * 16, 16)] = jnp.ones((16,), jnp.float32)
    pltpu.sync_copy(zb, acc.at[pl.ds(s * 640, 640)])
    plsc.subcore_barrier()

    ebase = w * _EPW_DEG

    def body(j, carry):
        off = pl.multiple_of(ebase + j * _CHUNK, _CHUNK)
        pltpu.sync_copy(dst_hbm.at[pl.ds(off, _CHUNK)], dstb)
        pltpu.sync_copy(onesb, acc.at[dstb], add=True)
        return carry

    lax.fori_loop(0, _ECHUNKS_DEG, body, 0)
    plsc.subcore_barrier()
    pltpu.sync_copy(acc.at[pl.ds(s * 640, 640)], zb)
    pltpu.sync_copy(zb, out_hbm.at[pl.ds(c * _NB_DEG + s * 640, 640)])


@functools.partial(
    pl.kernel,
    out_type=jax.ShapeDtypeStruct((2 * _N, _F), jnp.float32),
    mesh=_sc_mesh,
    scratch_types=[
        pltpu.VMEM((_CHUNK,), jnp.int32),
        pltpu.VMEM((_CHUNK,), jnp.int32),
        pltpu.VMEM((_CHUNK, _F), jnp.float32),
        pltpu.VMEM_SHARED((_NR, _F), jnp.float32),
        pltpu.SemaphoreType.DMA,
    ],
)
def _agg(g_hbm, src_hbm, dst_hbm, out_hbm, srcb, dstb, rows, acc, sem):
    c = lax.axis_index("c")
    s = lax.axis_index("s")
    rbase = s * _ROWS_PER_SUB
    for k in range(_ROWS_PER_SUB // _INIT_CHUNK):
        r0 = rbase + k * _INIT_CHUNK
        pltpu.sync_copy(g_hbm.at[pl.ds(c * _N + r0, _INIT_CHUNK)],
                        rows.at[pl.ds(0, _INIT_CHUNK)])
        pltpu.sync_copy(rows.at[pl.ds(0, _INIT_CHUNK)],
                        acc.at[pl.ds(r0, _INIT_CHUNK)])
    plsc.subcore_barrier()

    ebase = s * _EPW_AGG
    cN = c * _N

    def body(j, carry):
        off = pl.multiple_of(ebase + j * _CHUNK, _CHUNK)
        pltpu.sync_copy(src_hbm.at[pl.ds(off, _CHUNK)], srcb)
        pltpu.sync_copy(dst_hbm.at[pl.ds(off, _CHUNK)], dstb)
        for i in range(_CHUNK // 16):
            sl = pl.ds(i * 16, 16)
            srcb[sl] = srcb[sl] + cN
        pltpu.async_copy(g_hbm.at[srcb], rows, sem).wait()
        pltpu.sync_copy(rows, acc.at[dstb], add=True)
        return carry

    lax.fori_loop(0, _ECHUNKS_AGG, body, 0)
    plsc.subcore_barrier()
    for k in range(_ROWS_PER_SUB // _INIT_CHUNK):
        r0 = rbase + k * _INIT_CHUNK
        pltpu.sync_copy(acc.at[pl.ds(r0, _INIT_CHUNK)],
                        rows.at[pl.ds(0, _INIT_CHUNK)])
        pltpu.sync_copy(rows.at[pl.ds(0, _INIT_CHUNK)],
                        out_hbm.at[pl.ds(c * _N + r0, _INIT_CHUNK)])


def _dinv(d0, d1):
    return lax.rsqrt(d0 + d1 + 1.0)


def _k1_body(x_ref, w_ref, b_ref, d0_ref, d1_ref, out_ref):
    dinv = _dinv(d0_ref[...], d1_ref[...])
    h = jnp.dot(x_ref[...], w_ref[...], preferred_element_type=jnp.float32)
    out_ref[...] = (h + b_ref[0:1, :]) * dinv


def _k23_body(s0_ref, s1_ref, wa_ref, wb_ref, b_ref, d0_ref, d1_ref, out_ref):
    dinv = _dinv(d0_ref[...], d1_ref[...])
    xa = jax.nn.relu(dinv * s0_ref[...])
    xb = jax.nn.relu(dinv * s1_ref[...])
    h = (jnp.dot(xa, wa_ref[...], preferred_element_type=jnp.float32)
         + jnp.dot(xb, wb_ref[...], preferred_element_type=jnp.float32))
    out_ref[...] = (h + b_ref[0:1, :]) * dinv


def _kout_body(s1a, s1b, s2a, s2b, s3a, s3b, w_ref, b_ref, d0_ref, d1_ref,
               out_ref):
    dinv = _dinv(d0_ref[...], d1_ref[...])
    acc = jnp.broadcast_to(b_ref[0:1, :], out_ref.shape)
    for l, sref in enumerate((s1a, s1b, s2a, s2b, s3a, s3b)):
        xl = jax.nn.relu(dinv * sref[...])
        acc = acc + jnp.dot(xl, w_ref[pl.ds(l * _F, _F), :],
                            preferred_element_type=jnp.float32)
    out_ref[...] = acc


_row_spec = pl.BlockSpec((_BN, _F), lambda h, b: (b, 0))
_deg_spec = pl.BlockSpec((_BN, 1), lambda h, b: (b, 0))
_out2n_spec = pl.BlockSpec((_BN, _F), lambda h, b: (h * _NBLK + b, 0))

_k1 = pl.pallas_call(
    _k1_body,
    grid=(2, _NBLK),
    in_specs=[
        _row_spec,
        pl.BlockSpec((_F, _F), lambda h, b: (0, h)),
        pl.BlockSpec((8, _F), lambda h, b: (0, h)),
        _deg_spec,
        _deg_spec,
    ],
    out_specs=_out2n_spec,
    out_shape=jax.ShapeDtypeStruct((2 * _N, _F), jnp.float32),
)

_k23 = pl.pallas_call(
    _k23_body,
    grid=(2, _NBLK),
    in_specs=[
        _row_spec,
        _row_spec,
        pl.BlockSpec((_F, _F), lambda h, b: (0, h)),
        pl.BlockSpec((_F, _F), lambda h, b: (0, h)),
        pl.BlockSpec((8, _F), lambda h, b: (0, h)),
        _deg_spec,
        _deg_spec,
    ],
    out_specs=_out2n_spec,
    out_shape=jax.ShapeDtypeStruct((2 * _N, _F), jnp.float32),
)

_kout = pl.pallas_call(
    _kout_body,
    grid=(_NBLK,),
    in_specs=[pl.BlockSpec((_BN, _F), lambda b: (b, 0))] * 6
    + [
        pl.BlockSpec((6 * _F, _F), lambda b: (0, 0)),
        pl.BlockSpec((8, _F), lambda b: (0, 0)),
        pl.BlockSpec((_BN, 1), lambda b: (b, 0)),
        pl.BlockSpec((_BN, 1), lambda b: (b, 0)),
    ],
    out_specs=pl.BlockSpec((_BN, _F), lambda b: (b, 0)),
    out_shape=jax.ShapeDtypeStruct((_N, _F), jnp.float32),
)


def kernel(x, edge_index, W1, b1, W2, b2, W3, b3, Wout, bout):
    src = edge_index[0]
    dst = edge_index[1]
    npad = _EPAD - _E
    fill = jnp.arange(npad, dtype=jnp.int32)
    src_p = jnp.concatenate([src, fill % _N])
    dst_p = jnp.concatenate([dst, _N + (fill % 16)])

    deg2 = _deg(dst_p)
    d0 = deg2[:_N].reshape(_N, 1)
    d1 = deg2[_NB_DEG:_NB_DEG + _N].reshape(_N, 1)

    b1b = jnp.broadcast_to(b1, (8, _H))
    b2b = jnp.broadcast_to(b2, (8, _H))
    b3b = jnp.broadcast_to(b3, (8, _H))
    wout_p = jnp.pad(Wout, ((0, 0), (0, _F - _C)))
    bout_p = jnp.broadcast_to(jnp.pad(bout, (0, _F - _C)), (8, _F))

    g1 = _k1(x, W1, b1b, d0, d1)
    s1 = _agg(g1, src_p, dst_p)
    g2 = _k23(s1[:_N], s1[_N:], W2[:_F], W2[_F:], b2b, d0, d1)
    s2 = _agg(g2, src_p, dst_p)
    g3 = _k23(s2[:_N], s2[_N:], W3[:_F], W3[_F:], b3b, d0, d1)
    s3 = _agg(g3, src_p, dst_p)

    out = _kout(s1[:_N], s1[_N:], s2[:_N], s2[_N:], s3[:_N], s3[_N:],
                wout_p, bout_p, d0, d1)
    return out[:, :_C]


# speedup vs baseline: 9.4240x; 9.4240x over previous
import functools

import jax
import jax.numpy as jnp
from jax import lax
from jax.experimental import pallas as pl
from jax.experimental.pallas import tpu as pltpu
from jax.experimental.pallas import tpu_sc as plsc

_N = 10000
_E = 320000
_F = 128
_H = 256
_C = 40

_NC = 2
_NS = 16
_CHUNK = 128
_EPAD = 323584
_ECHUNKS_AGG = _EPAD // (_NS * _CHUNK)
_EPW_AGG = _EPAD // _NS
_ECHUNKS_DEG = _EPAD // (_NC * _NS * _CHUNK)
_EPW_DEG = _EPAD // (_NC * _NS)
_NP = 10240
_NR = _NP + 16
_NB_DEG = 10240
_ROWS_PER_SUB = _NP // _NS
_INIT_CHUNK = 128

_BN = 1024
_NBLK = _NP // _BN


_sc_mesh = plsc.VectorSubcoreMesh(core_axis_name="c", subcore_axis_name="s")


@functools.partial(
    pl.kernel,
    out_type=jax.ShapeDtypeStruct((_NC * _NB_DEG,), jnp.float32),
    mesh=_sc_mesh,
    scratch_types=[
        pltpu.VMEM((_CHUNK,), jnp.int32),
        pltpu.VMEM((_CHUNK,), jnp.float32),
        pltpu.VMEM((640,), jnp.float32),
        pltpu.VMEM_SHARED((_NB_DEG,), jnp.float32),
    ],
)
def _deg(dst_hbm, out_hbm, dstb, onesb, zb, acc):
    c = lax.axis_index("c")
    s = lax.axis_index("s")
    w = c * _NS + s
    for i in range(640 // 16):
        zb[pl.ds(i * 16, 16)] = jnp.zeros((16,), jnp.float32)
    for i in range(_CHUNK // 16):
        onesb[pl.ds(i * 16, 16)] = jnp.ones((16,), jnp.float32)
    pltpu.sync_copy(zb, acc.at[pl.ds(s * 640, 640)])
    plsc.subcore_barrier()

    ebase = w * _EPW_DEG

    def body(j, carry):
        off = pl.multiple_of(ebase + j * _CHUNK, _CHUNK)
        pltpu.sync_copy(dst_hbm.at[pl.ds(off, _CHUNK)], dstb)
        pltpu.sync_copy(onesb, acc.at[dstb], add=True)
        return carry

    lax.fori_loop(0, _ECHUNKS_DEG, body, 0)
    plsc.subcore_barrier()
    pltpu.sync_copy(acc.at[pl.ds(s * 640, 640)], zb)
    pltpu.sync_copy(zb, out_hbm.at[pl.ds(c * _NB_DEG + s * 640, 640)])


@functools.partial(
    pl.kernel,
    out_type=jax.ShapeDtypeStruct((2 * _NP, _F), jnp.float32),
    mesh=_sc_mesh,
    scratch_types=[
        pltpu.VMEM((_CHUNK,), jnp.int32),
        pltpu.VMEM((_CHUNK,), jnp.int32),
        pltpu.VMEM((_CHUNK, _F), jnp.float32),
        pltpu.VMEM_SHARED((_NR, _F), jnp.float32),
        pltpu.SemaphoreType.DMA,
    ],
)
def _agg(g_hbm, src_hbm, dst_hbm, out_hbm, srcb, dstb, rows, acc, sem):
    c = lax.axis_index("c")
    s = lax.axis_index("s")
    rbase = s * _ROWS_PER_SUB
    for k in range(_ROWS_PER_SUB // _INIT_CHUNK):
        r0 = rbase + k * _INIT_CHUNK
        pltpu.sync_copy(g_hbm.at[pl.ds(c * _NP + r0, _INIT_CHUNK)], rows)
        pltpu.sync_copy(rows, acc.at[pl.ds(r0, _INIT_CHUNK)])
    plsc.subcore_barrier()

    ebase = s * _EPW_AGG
    cN = c * _NP

    def body(j, carry):
        off = pl.multiple_of(ebase + j * _CHUNK, _CHUNK)
        pltpu.sync_copy(src_hbm.at[pl.ds(off, _CHUNK)], srcb)
        pltpu.sync_copy(dst_hbm.at[pl.ds(off, _CHUNK)], dstb)
        for i in range(_CHUNK // 16):
            sl = pl.ds(i * 16, 16)
            srcb[sl] = srcb[sl] + cN
        pltpu.async_copy(g_hbm.at[srcb], rows, sem).wait()
        pltpu.sync_copy(rows, acc.at[dstb], add=True)
        return carry

    lax.fori_loop(0, _ECHUNKS_AGG, body, 0)
    plsc.subcore_barrier()
    for k in range(_ROWS_PER_SUB // _INIT_CHUNK):
        r0 = rbase + k * _INIT_CHUNK
        pltpu.sync_copy(acc.at[pl.ds(r0, _INIT_CHUNK)], rows)
        pltpu.sync_copy(rows, out_hbm.at[pl.ds(c * _NP + r0, _INIT_CHUNK)])


def _dinv(d0, d1):
    return lax.rsqrt(d0 + d1 + 1.0)


def _k1_body(x_ref, w_ref, b_ref, d0_ref, d1_ref, out_ref):
    dinv = _dinv(d0_ref[...], d1_ref[...])
    h = jnp.dot(x_ref[...], w_ref[...], preferred_element_type=jnp.float32)
    out_ref[...] = (h + b_ref[0:1, :]) * dinv


def _k23_body(s0_ref, s1_ref, wa_ref, wb_ref, b_ref, d0_ref, d1_ref, out_ref):
    dinv = _dinv(d0_ref[...], d1_ref[...])
    xa = jax.nn.relu(dinv * s0_ref[...])
    xb = jax.nn.relu(dinv * s1_ref[...])
    h = (jnp.dot(xa, wa_ref[...], preferred_element_type=jnp.float32)
         + jnp.dot(xb, wb_ref[...], preferred_element_type=jnp.float32))
    out_ref[...] = (h + b_ref[0:1, :]) * dinv


def _kout_body(s1a, s1b, s2a, s2b, s3a, s3b, w_ref, b_ref, d0_ref, d1_ref,
               out_ref):
    dinv = _dinv(d0_ref[...], d1_ref[...])
    acc = jnp.broadcast_to(b_ref[0:1, :], out_ref.shape)
    for l, sref in enumerate((s1a, s1b, s2a, s2b, s3a, s3b)):
        xl = jax.nn.relu(dinv * sref[...])
        acc = acc + jnp.dot(xl, w_ref[pl.ds(l * _F, _F), :],
                            preferred_element_type=jnp.float32)
    out_ref[...] = acc


_row_spec = pl.BlockSpec((_BN, _F), lambda h, b: (b, 0))
_deg_spec = pl.BlockSpec((_BN, 1), lambda h, b: (b, 0))
_out2n_spec = pl.BlockSpec((_BN, _F), lambda h, b: (h * _NBLK + b, 0))

_k1 = pl.pallas_call(
    _k1_body,
    grid=(2, _NBLK),
    in_specs=[
        _row_spec,
        pl.BlockSpec((_F, _F), lambda h, b: (0, h)),
        pl.BlockSpec((8, _F), lambda h, b: (0, h)),
        _deg_spec,
        _deg_spec,
    ],
    out_specs=_out2n_spec,
    out_shape=jax.ShapeDtypeStruct((2 * _NP, _F), jnp.float32),
)

_k23 = pl.pallas_call(
    _k23_body,
    grid=(2, _NBLK),
    in_specs=[
        _row_spec,
        _row_spec,
        pl.BlockSpec((_F, _F), lambda h, b: (0, h)),
        pl.BlockSpec((_F, _F), lambda h, b: (0, h)),
        pl.BlockSpec((8, _F), lambda h, b: (0, h)),
        _deg_spec,
        _deg_spec,
    ],
    out_specs=_out2n_spec,
    out_shape=jax.ShapeDtypeStruct((2 * _NP, _F), jnp.float32),
)

_kout = pl.pallas_call(
    _kout_body,
    grid=(_NBLK,),
    in_specs=[pl.BlockSpec((_BN, _F), lambda b: (b, 0))] * 6
    + [
        pl.BlockSpec((6 * _F, _F), lambda b: (0, 0)),
        pl.BlockSpec((8, _F), lambda b: (0, 0)),
        pl.BlockSpec((_BN, 1), lambda b: (b, 0)),
        pl.BlockSpec((_BN, 1), lambda b: (b, 0)),
    ],
    out_specs=pl.BlockSpec((_BN, _F), lambda b: (b, 0)),
    out_shape=jax.ShapeDtypeStruct((_NP, _F), jnp.float32),
)


def kernel(x, edge_index, W1, b1, W2, b2, W3, b3, Wout, bout):
    src = edge_index[0]
    dst = edge_index[1]
    npad = _EPAD - _E
    fill = jnp.arange(npad, dtype=jnp.int32)
    src_p = jnp.concatenate([src, fill % _N])
    dst_p = jnp.concatenate([dst, _NP + (fill % 16)])

    deg2 = _deg(dst_p)
    d0 = deg2[:_NP].reshape(_NP, 1)
    d1 = deg2[_NB_DEG:].reshape(_NP, 1)

    b1b = jnp.broadcast_to(b1, (8, _H))
    b2b = jnp.broadcast_to(b2, (8, _H))
    b3b = jnp.broadcast_to(b3, (8, _H))
    wout_p = jnp.pad(Wout, ((0, 0), (0, _F - _C)))
    bout_p = jnp.broadcast_to(jnp.pad(bout, (0, _F - _C)), (8, _F))

    x_p = jnp.pad(x, ((0, _NP - _N), (0, 0)))
    g1 = _k1(x_p, W1, b1b, d0, d1)
    s1 = _agg(g1, src_p, dst_p)
    g2 = _k23(s1[:_NP], s1[_NP:], W2[:_F], W2[_F:], b2b, d0, d1)
    s2 = _agg(g2, src_p, dst_p)
    g3 = _k23(s2[:_NP], s2[_NP:], W3[:_F], W3[_F:], b3b, d0, d1)
    s3 = _agg(g3, src_p, dst_p)

    out = _kout(s1[:_NP], s1[_NP:], s2[:_NP], s2[_NP:], s3[:_NP], s3[_NP:],
                wout_p, bout_p, d0, d1)
    return out[:_N, :_C]


# speedup vs baseline: 15.7418x; 1.6704x over previous
import functools

import jax
import jax.numpy as jnp
from jax import lax
from jax.experimental import pallas as pl
from jax.experimental.pallas import tpu as pltpu
from jax.experimental.pallas import tpu_sc as plsc

_N = 10000
_E = 320000
_F = 128
_H = 256
_C = 40

_NC = 2
_NS = 16
_CHUNK = 128
_EPAD = 323584
_ECHUNKS_AGG = _EPAD // (_NS * _CHUNK)
_EPW_AGG = _EPAD // _NS
_ECHUNKS_DEG = _EPAD // (_NC * _NS * _CHUNK)
_EPW_DEG = _EPAD // (_NC * _NS)
_NP = 10240
_NR = _NP + 16
_NB_DEG = 10240
_ROWS_PER_SUB = _NP // _NS
_INIT_CHUNK = 128

_BN = 1024
_NBLK = _NP // _BN


_sc_mesh = plsc.VectorSubcoreMesh(core_axis_name="c", subcore_axis_name="s")


@functools.partial(
    pl.kernel,
    out_type=jax.ShapeDtypeStruct((_NC * _NB_DEG,), jnp.float32),
    mesh=_sc_mesh,
    scratch_types=[
        pltpu.VMEM((_CHUNK,), jnp.int32),
        pltpu.VMEM((_CHUNK,), jnp.float32),
        pltpu.VMEM((640,), jnp.float32),
        pltpu.VMEM_SHARED((_NB_DEG,), jnp.float32),
    ],
)
def _deg(dst_hbm, out_hbm, dstb, onesb, zb, acc):
    c = lax.axis_index("c")
    s = lax.axis_index("s")
    w = c * _NS + s
    for i in range(640 // 16):
        zb[pl.ds(i * 16, 16)] = jnp.zeros((16,), jnp.float32)
    for i in range(_CHUNK // 16):
        onesb[pl.ds(i * 16, 16)] = jnp.ones((16,), jnp.float32)
    pltpu.sync_copy(zb, acc.at[pl.ds(s * 640, 640)])
    plsc.subcore_barrier()

    ebase = w * _EPW_DEG

    def body(j, carry):
        off = pl.multiple_of(ebase + j * _CHUNK, _CHUNK)
        pltpu.sync_copy(dst_hbm.at[pl.ds(off, _CHUNK)], dstb)
        pltpu.sync_copy(onesb, acc.at[dstb], add=True)
        return carry

    lax.fori_loop(0, _ECHUNKS_DEG, body, 0)
    plsc.subcore_barrier()
    pltpu.sync_copy(acc.at[pl.ds(s * 640, 640)], zb)
    pltpu.sync_copy(zb, out_hbm.at[pl.ds(c * _NB_DEG + s * 640, 640)])


@functools.partial(
    pl.kernel,
    out_type=jax.ShapeDtypeStruct((2 * _NP, _F), jnp.float32),
    mesh=_sc_mesh,
    scratch_types=[
        pltpu.VMEM((_CHUNK,), jnp.int32),
        pltpu.VMEM((_CHUNK,), jnp.int32),
        pltpu.VMEM((_CHUNK,), jnp.int32),
        pltpu.VMEM((_CHUNK,), jnp.int32),
        pltpu.VMEM((_CHUNK, _F), jnp.float32),
        pltpu.VMEM((_CHUNK, _F), jnp.float32),
        pltpu.VMEM_SHARED((_NR, _F), jnp.float32),
        pltpu.SemaphoreType.DMA,
        pltpu.SemaphoreType.DMA,
        pltpu.SemaphoreType.DMA,
        pltpu.SemaphoreType.DMA,
    ],
)
def _agg(g_hbm, src_hbm, dst_hbm, out_hbm, sb0, sb1, db0, db1, rows0, rows1,
         acc, is0, is1, rs0, rs1):
    c = lax.axis_index("c")
    s = lax.axis_index("s")
    rbase = s * _ROWS_PER_SUB
    for k in range(_ROWS_PER_SUB // _INIT_CHUNK):
        r0 = rbase + k * _INIT_CHUNK
        pltpu.sync_copy(g_hbm.at[pl.ds(c * _NP + r0, _INIT_CHUNK)], rows0)
        pltpu.sync_copy(rows0, acc.at[pl.ds(r0, _INIT_CHUNK)])
    plsc.subcore_barrier()

    sbase = c * _EPAD + s * _EPW_AGG
    dbase = s * _EPW_AGG

    def iload(j, sb, db, sem):
        off = pl.multiple_of(j * _CHUNK, _CHUNK)
        pltpu.async_copy(src_hbm.at[pl.ds(sbase + off, _CHUNK)], sb, sem)
        pltpu.async_copy(dst_hbm.at[pl.ds(dbase + off, _CHUNK)], db, sem)

    def iwait(sb, db, sem):
        pltpu.make_async_copy(src_hbm.at[pl.ds(0, _CHUNK)], sb, sem).wait()
        pltpu.make_async_copy(dst_hbm.at[pl.ds(0, _CHUNK)], db, sem).wait()

    def gwait(rows, sem):
        pltpu.make_async_copy(g_hbm.at[pl.ds(0, _CHUNK)], rows, sem).wait()

    iload(0, sb0, db0, is0)
    iload(1, sb1, db1, is1)
    iwait(sb0, db0, is0)
    pltpu.async_copy(g_hbm.at[sb0], rows0, rs0)

    def pair(jp, prefetch):
        j = 2 * jp
        iwait(sb1, db1, is1)
        pltpu.async_copy(g_hbm.at[sb1], rows1, rs1)
        gwait(rows0, rs0)
        pltpu.sync_copy(rows0, acc.at[db0], add=True)
        if prefetch:
            iload(j + 2, sb0, db0, is0)
        gwait(rows1, rs1)
        pltpu.sync_copy(rows1, acc.at[db1], add=True)
        if prefetch:
            iload(j + 3, sb1, db1, is1)
            iwait(sb0, db0, is0)
            pltpu.async_copy(g_hbm.at[sb0], rows0, rs0)

    def body(jp, carry):
        pair(jp, True)
        return carry

    lax.fori_loop(0, _ECHUNKS_AGG // 2 - 1, body, 0)
    pair(_ECHUNKS_AGG // 2 - 1, False)
    plsc.subcore_barrier()
    for k in range(_ROWS_PER_SUB // _INIT_CHUNK):
        r0 = rbase + k * _INIT_CHUNK
        pltpu.sync_copy(acc.at[pl.ds(r0, _INIT_CHUNK)], rows0)
        pltpu.sync_copy(rows0, out_hbm.at[pl.ds(c * _NP + r0, _INIT_CHUNK)])


def _dinv(d0, d1):
    return lax.rsqrt(d0 + d1 + 1.0)


def _k1_body(x_ref, w_ref, b_ref, d0_ref, d1_ref, out_ref):
    dinv = _dinv(d0_ref[...], d1_ref[...])
    h = jnp.dot(x_ref[...], w_ref[...], preferred_element_type=jnp.float32)
    out_ref[...] = (h + b_ref[0:1, :]) * dinv


def _k23_body(s0_ref, s1_ref, wa_ref, wb_ref, b_ref, d0_ref, d1_ref, out_ref):
    dinv = _dinv(d0_ref[...], d1_ref[...])
    xa = jax.nn.relu(dinv * s0_ref[...])
    xb = jax.nn.relu(dinv * s1_ref[...])
    h = (jnp.dot(xa, wa_ref[...], preferred_element_type=jnp.float32)
         + jnp.dot(xb, wb_ref[...], preferred_element_type=jnp.float32))
    out_ref[...] = (h + b_ref[0:1, :]) * dinv


def _kout_body(s1a, s1b, s2a, s2b, s3a, s3b, w_ref, b_ref, d0_ref, d1_ref,
               out_ref):
    dinv = _dinv(d0_ref[...], d1_ref[...])
    acc = jnp.broadcast_to(b_ref[0:1, :], out_ref.shape)
    for l, sref in enumerate((s1a, s1b, s2a, s2b, s3a, s3b)):
        xl = jax.nn.relu(dinv * sref[...])
        acc = acc + jnp.dot(xl, w_ref[pl.ds(l * _F, _F), :],
                            preferred_element_type=jnp.float32)
    out_ref[...] = acc


_row_spec = pl.BlockSpec((_BN, _F), lambda h, b: (b, 0))
_deg_spec = pl.BlockSpec((_BN, 1), lambda h, b: (b, 0))
_out2n_spec = pl.BlockSpec((_BN, _F), lambda h, b: (h * _NBLK + b, 0))

_k1 = pl.pallas_call(
    _k1_body,
    grid=(2, _NBLK),
    in_specs=[
        _row_spec,
        pl.BlockSpec((_F, _F), lambda h, b: (0, h)),
        pl.BlockSpec((8, _F), lambda h, b: (0, h)),
        _deg_spec,
        _deg_spec,
    ],
    out_specs=_out2n_spec,
    out_shape=jax.ShapeDtypeStruct((2 * _NP, _F), jnp.float32),
)

_k23 = pl.pallas_call(
    _k23_body,
    grid=(2, _NBLK),
    in_specs=[
        _row_spec,
        _row_spec,
        pl.BlockSpec((_F, _F), lambda h, b: (0, h)),
        pl.BlockSpec((_F, _F), lambda h, b: (0, h)),
        pl.BlockSpec((8, _F), lambda h, b: (0, h)),
        _deg_spec,
        _deg_spec,
    ],
    out_specs=_out2n_spec,
    out_shape=jax.ShapeDtypeStruct((2 * _NP, _F), jnp.float32),
)

_kout = pl.pallas_call(
    _kout_body,
    grid=(_NBLK,),
    in_specs=[pl.BlockSpec((_BN, _F), lambda b: (b, 0))] * 6
    + [
        pl.BlockSpec((6 * _F, _F), lambda b: (0, 0)),
        pl.BlockSpec((8, _F), lambda b: (0, 0)),
        pl.BlockSpec((_BN, 1), lambda b: (b, 0)),
        pl.BlockSpec((_BN, 1), lambda b: (b, 0)),
    ],
    out_specs=pl.BlockSpec((_BN, _F), lambda b: (b, 0)),
    out_shape=jax.ShapeDtypeStruct((_NP, _F), jnp.float32),
)


def kernel(x, edge_index, W1, b1, W2, b2, W3, b3, Wout, bout):
    src = edge_index[0]
    dst = edge_index[1]
    npad = _EPAD - _E
    fill = jnp.arange(npad, dtype=jnp.int32)
    src_p = jnp.concatenate([src, fill % _N])
    dst_p = jnp.concatenate([dst, _NP + (fill % 16)])
    src_sh = jnp.concatenate([src_p, src_p + _NP])

    deg2 = _deg(dst_p)
    d0 = deg2[:_NP].reshape(_NP, 1)
    d1 = deg2[_NB_DEG:].reshape(_NP, 1)

    b1b = jnp.broadcast_to(b1, (8, _H))
    b2b = jnp.broadcast_to(b2, (8, _H))
    b3b = jnp.broadcast_to(b3, (8, _H))
    wout_p = jnp.pad(Wout, ((0, 0), (0, _F - _C)))
    bout_p = jnp.broadcast_to(jnp.pad(bout, (0, _F - _C)), (8, _F))

    x_p = jnp.pad(x, ((0, _NP - _N), (0, 0)))
    g1 = _k1(x_p, W1, b1b, d0, d1)
    s1 = _agg(g1, src_sh, dst_p)
    g2 = _k23(s1[:_NP], s1[_NP:], W2[:_F], W2[_F:], b2b, d0, d1)
    s2 = _agg(g2, src_sh, dst_p)
    g3 = _k23(s2[:_NP], s2[_NP:], W3[:_F], W3[_F:], b3b, d0, d1)
    s3 = _agg(g3, src_sh, dst_p)

    out = _kout(s1[:_NP], s1[_NP:], s2[:_NP], s2[_NP:], s3[:_NP], s3[_NP:],
                wout_p, bout_p, d0, d1)
    return out[:_N, :_C]


# speedup vs baseline: 16.5239x; 1.0497x over previous
import functools

import jax
import jax.numpy as jnp
from jax import lax
from jax.experimental import pallas as pl
from jax.experimental.pallas import tpu as pltpu
from jax.experimental.pallas import tpu_sc as plsc

_N = 10000
_E = 320000
_F = 128
_H = 256
_C = 40

_NC = 2
_NS = 16
_CHUNK = 128
_EPAD = 323584
_ECHUNKS_AGG = _EPAD // (_NS * _CHUNK)
_EPW_AGG = _EPAD // _NS
_ECHUNKS_DEG = _EPAD // (_NC * _NS * _CHUNK)
_EPW_DEG = _EPAD // (_NC * _NS)
_NP = 10240
_NR = _NP + 16
_NB_DEG = 10240
_ROWS_PER_SUB = _NP // _NS
_INIT_CHUNK = 128

_BN = 1024
_NBLK = _NP // _BN


_sc_mesh = plsc.VectorSubcoreMesh(core_axis_name="c", subcore_axis_name="s")


@functools.partial(
    pl.kernel,
    out_type=jax.ShapeDtypeStruct((_NC * _NB_DEG,), jnp.float32),
    mesh=_sc_mesh,
    scratch_types=[
        pltpu.VMEM((_CHUNK,), jnp.int32),
        pltpu.VMEM((_CHUNK,), jnp.float32),
        pltpu.VMEM((640,), jnp.float32),
        pltpu.VMEM_SHARED((_NB_DEG,), jnp.float32),
    ],
)
def _deg(dst_hbm, out_hbm, dstb, onesb, zb, acc):
    c = lax.axis_index("c")
    s = lax.axis_index("s")
    w = c * _NS + s
    for i in range(640 // 16):
        zb[pl.ds(i * 16, 16)] = jnp.zeros((16,), jnp.float32)
    for i in range(_CHUNK // 16):
        onesb[pl.ds(i * 16, 16)] = jnp.ones((16,), jnp.float32)
    pltpu.sync_copy(zb, acc.at[pl.ds(s * 640, 640)])
    plsc.subcore_barrier()

    ebase = w * _EPW_DEG

    def body(j, carry):
        off = pl.multiple_of(ebase + j * _CHUNK, _CHUNK)
        pltpu.sync_copy(dst_hbm.at[pl.ds(off, _CHUNK)], dstb)
        pltpu.sync_copy(onesb, acc.at[dstb], add=True)
        return carry

    lax.fori_loop(0, _ECHUNKS_DEG, body, 0)
    plsc.subcore_barrier()
    pltpu.sync_copy(acc.at[pl.ds(s * 640, 640)], zb)
    pltpu.sync_copy(zb, out_hbm.at[pl.ds(c * _NB_DEG + s * 640, 640)])


@functools.partial(
    pl.kernel,
    out_type=jax.ShapeDtypeStruct((2 * _NP, _F), jnp.float32),
    mesh=_sc_mesh,
    scratch_types=[
        pltpu.VMEM((4, _CHUNK), jnp.int32),
        pltpu.VMEM((4, _CHUNK), jnp.int32),
        pltpu.VMEM((_CHUNK, _F), jnp.float32),
        pltpu.VMEM((_CHUNK, _F), jnp.float32),
        pltpu.VMEM_SHARED((_NR, _F), jnp.float32),
        pltpu.SemaphoreType.DMA,
        pltpu.SemaphoreType.DMA,
        pltpu.SemaphoreType.DMA,
        pltpu.SemaphoreType.DMA,
        pltpu.SemaphoreType.DMA,
        pltpu.SemaphoreType.DMA,
        pltpu.SemaphoreType.DMA,
        pltpu.SemaphoreType.DMA,
    ],
)
def _agg(g_hbm, src_hbm, dst_hbm, out_hbm, sbs, dbs, rows0, rows1,
         acc, i0, i1, i2, i3, r0s, r1s, s0s, s1s):
    c = lax.axis_index("c")
    s = lax.axis_index("s")
    rbase = s * _ROWS_PER_SUB
    for k in range(_ROWS_PER_SUB // _INIT_CHUNK):
        r0 = rbase + k * _INIT_CHUNK
        pltpu.sync_copy(g_hbm.at[pl.ds(c * _NP + r0, _INIT_CHUNK)], rows0)
        pltpu.sync_copy(rows0, acc.at[pl.ds(r0, _INIT_CHUNK)])
    plsc.subcore_barrier()

    sbase = c * _EPAD + s * _EPW_AGG
    dbase = s * _EPW_AGG
    isem = (i0, i1, i2, i3)
    rsem = (r0s, r1s)
    ssem = (s0s, s1s)
    rows = (rows0, rows1)

    def iload(j, q):
        off = pl.multiple_of(j * _CHUNK, _CHUNK)
        pltpu.async_copy(src_hbm.at[pl.ds(sbase + off, _CHUNK)],
                         sbs.at[q], isem[q])
        pltpu.async_copy(dst_hbm.at[pl.ds(dbase + off, _CHUNK)],
                         dbs.at[q], isem[q])

    def iwait(q):
        pltpu.make_async_copy(src_hbm.at[pl.ds(0, _CHUNK)], sbs.at[q],
                              isem[q]).wait()
        pltpu.make_async_copy(dst_hbm.at[pl.ds(0, _CHUNK)], dbs.at[q],
                              isem[q]).wait()

    def emit(j, q, b, first, has_next, do_iload):
        pltpu.make_async_copy(g_hbm.at[pl.ds(0, _CHUNK)], rows[b],
                              rsem[b]).wait()
        pltpu.async_copy(rows[b], acc.at[dbs.at[q]], ssem[b], add=True)
        if has_next:
            if not first:
                pltpu.make_async_copy(rows[1 - b], acc.at[dbs.at[q]],
                                      ssem[1 - b]).wait()
            qn = (q + 1) % 4
            iwait(qn)
            pltpu.async_copy(g_hbm.at[sbs.at[qn]], rows[1 - b], rsem[1 - b])
        if do_iload:
            iload(j + 3, (q + 3) % 4)

    iload(0, 0)
    iload(1, 1)
    iload(2, 2)
    iwait(0)
    pltpu.async_copy(g_hbm.at[sbs.at[0]], rows0, r0s)

    emit(0, 0, 0, True, True, True)
    emit(1, 1, 1, False, True, True)
    emit(2, 2, 0, False, True, True)
    emit(3, 3, 1, False, True, True)

    def body(i, carry):
        j = 4 * i
        emit(j + 0, 0, 0, False, True, True)
        emit(j + 1, 1, 1, False, True, True)
        emit(j + 2, 2, 0, False, True, True)
        emit(j + 3, 3, 1, False, True, True)
        return carry

    lax.fori_loop(1, 38, body, 0)
    emit(152, 0, 0, False, True, True)
    emit(153, 1, 1, False, True, True)
    emit(154, 2, 0, False, True, True)
    emit(155, 3, 1, False, True, False)
    emit(156, 0, 0, False, True, False)
    emit(157, 1, 1, False, False, False)
    pltpu.make_async_copy(rows0, acc.at[dbs.at[0]], s0s).wait()
    pltpu.make_async_copy(rows1, acc.at[dbs.at[1]], s1s).wait()
    plsc.subcore_barrier()
    for k in range(_ROWS_PER_SUB // _INIT_CHUNK):
        r0 = rbase + k * _INIT_CHUNK
        pltpu.sync_copy(acc.at[pl.ds(r0, _INIT_CHUNK)], rows0)
        pltpu.sync_copy(rows0, out_hbm.at[pl.ds(c * _NP + r0, _INIT_CHUNK)])


def _dinv(d0, d1):
    return lax.rsqrt(d0 + d1 + 1.0)


def _k1_body(x_ref, w_ref, b_ref, d0_ref, d1_ref, out_ref):
    dinv = _dinv(d0_ref[...], d1_ref[...])
    h = jnp.dot(x_ref[...], w_ref[...], preferred_element_type=jnp.float32)
    out_ref[...] = (h + b_ref[0:1, :]) * dinv


def _k23_body(s0_ref, s1_ref, wa_ref, wb_ref, b_ref, d0_ref, d1_ref, out_ref):
    dinv = _dinv(d0_ref[...], d1_ref[...])
    xa = jax.nn.relu(dinv * s0_ref[...])
    xb = jax.nn.relu(dinv * s1_ref[...])
    h = (jnp.dot(xa, wa_ref[...], preferred_element_type=jnp.float32)
         + jnp.dot(xb, wb_ref[...], preferred_element_type=jnp.float32))
    out_ref[...] = (h + b_ref[0:1, :]) * dinv


def _kout_body(s1a, s1b, s2a, s2b, s3a, s3b, w_ref, b_ref, d0_ref, d1_ref,
               out_ref):
    dinv = _dinv(d0_ref[...], d1_ref[...])
    acc = jnp.broadcast_to(b_ref[0:1, :], out_ref.shape)
    for l, sref in enumerate((s1a, s1b, s2a, s2b, s3a, s3b)):
        xl = jax.nn.relu(dinv * sref[...])
        acc = acc + jnp.dot(xl, w_ref[pl.ds(l * _F, _F), :],
                            preferred_element_type=jnp.float32)
    out_ref[...] = acc


_row_spec = pl.BlockSpec((_BN, _F), lambda h, b: (b, 0))
_deg_spec = pl.BlockSpec((_BN, 1), lambda h, b: (b, 0))
_out2n_spec = pl.BlockSpec((_BN, _F), lambda h, b: (h * _NBLK + b, 0))

_k1 = pl.pallas_call(
    _k1_body,
    grid=(2, _NBLK),
    in_specs=[
        _row_spec,
        pl.BlockSpec((_F, _F), lambda h, b: (0, h)),
        pl.BlockSpec((8, _F), lambda h, b: (0, h)),
        _deg_spec,
        _deg_spec,
    ],
    out_specs=_out2n_spec,
    out_shape=jax.ShapeDtypeStruct((2 * _NP, _F), jnp.float32),
)

_k23 = pl.pallas_call(
    _k23_body,
    grid=(2, _NBLK),
    in_specs=[
        _row_spec,
        _row_spec,
        pl.BlockSpec((_F, _F), lambda h, b: (0, h)),
        pl.BlockSpec((_F, _F), lambda h, b: (0, h)),
        pl.BlockSpec((8, _F), lambda h, b: (0, h)),
        _deg_spec,
        _deg_spec,
    ],
    out_specs=_out2n_spec,
    out_shape=jax.ShapeDtypeStruct((2 * _NP, _F), jnp.float32),
)

_kout = pl.pallas_call(
    _kout_body,
    grid=(_NBLK,),
    in_specs=[pl.BlockSpec((_BN, _F), lambda b: (b, 0))] * 6
    + [
        pl.BlockSpec((6 * _F, _F), lambda b: (0, 0)),
        pl.BlockSpec((8, _F), lambda b: (0, 0)),
        pl.BlockSpec((_BN, 1), lambda b: (b, 0)),
        pl.BlockSpec((_BN, 1), lambda b: (b, 0)),
    ],
    out_specs=pl.BlockSpec((_BN, _F), lambda b: (b, 0)),
    out_shape=jax.ShapeDtypeStruct((_NP, _F), jnp.float32),
)


def kernel(x, edge_index, W1, b1, W2, b2, W3, b3, Wout, bout):
    src = edge_index[0]
    dst = edge_index[1]
    npad = _EPAD - _E
    fill = jnp.arange(npad, dtype=jnp.int32)
    src_p = jnp.concatenate([src, fill % _N])
    dst_p = jnp.concatenate([dst, _NP + (fill % 16)])
    src_sh = jnp.concatenate([src_p, src_p + _NP])

    deg2 = _deg(dst_p)
    d0 = deg2[:_NP].reshape(_NP, 1)
    d1 = deg2[_NB_DEG:].reshape(_NP, 1)

    b1b = jnp.broadcast_to(b1, (8, _H))
    b2b = jnp.broadcast_to(b2, (8, _H))
    b3b = jnp.broadcast_to(b3, (8, _H))
    wout_p = jnp.pad(Wout, ((0, 0), (0, _F - _C)))
    bout_p = jnp.broadcast_to(jnp.pad(bout, (0, _F - _C)), (8, _F))

    x_p = jnp.pad(x, ((0, _NP - _N), (0, 0)))
    g1 = _k1(x_p, W1, b1b, d0, d1)
    s1 = _agg(g1, src_sh, dst_p)
    g2 = _k23(s1[:_NP], s1[_NP:], W2[:_F], W2[_F:], b2b, d0, d1)
    s2 = _agg(g2, src_sh, dst_p)
    g3 = _k23(s2[:_NP], s2[_NP:], W3[:_F], W3[_F:], b3b, d0, d1)
    s3 = _agg(g3, src_sh, dst_p)

    out = _kout(s1[:_NP], s1[_NP:], s2[:_NP], s2[_NP:], s3[:_NP], s3[_NP:],
                wout_p, bout_p, d0, d1)
    return out[:_N, :_C]
